# baseline (speedup 1.0000x reference)
import jax
import jax.numpy as jnp
from jax import lax
from jax.experimental import pallas as pl
from jax.experimental.pallas import tpu as pltpu

_TK = 2048
_L = 128


def _vq_kernel(x_ref, p_ref, cb_ref, out_ref):
    rows = x_ref.shape[0]
    k_total = cb_ref.shape[0]
    cd = cb_ref.shape[1]

    h = jnp.dot(x_ref[...], p_ref[...], preferred_element_type=jnp.float32)
    n = jnp.sqrt(jnp.sum(h * h, axis=1, keepdims=True))
    hn = h / jnp.maximum(n, 1e-12)

    ii = lax.broadcasted_iota(jnp.int32, (cd, cd), 0)
    jj = lax.broadcasted_iota(jnp.int32, (cd, cd), 1)
    eye = jnp.where(ii == jj, 1.0, 0.0).astype(jnp.float32)
    cbt_all = lax.dot_general(eye, cb_ref[...], (((1,), (1,)), ((), ())),
                              preferred_element_type=jnp.float32,
                              precision=lax.Precision.HIGHEST)

    mval = jnp.full((rows, _L), jnp.inf, dtype=jnp.float32)
    mcol = jnp.zeros((rows, _L), dtype=jnp.int32)
    for t in range(k_total // _TK):
        cbt = cbt_all[:, t * _TK:(t + 1) * _TK]
        cb2 = jnp.sum(cbt * cbt, axis=0, keepdims=True)
        cbt2 = cbt * (-2.0)
        s = jnp.dot(hn, cbt2, preferred_element_type=jnp.float32,
                    precision=lax.Precision.HIGHEST)
        d = cb2 + s
        for c in range(_TK // _L):
            dc = d[:, c * _L:(c + 1) * _L]
            upd = dc < mval
            mval = jnp.where(upd, dc, mval)
            mcol = jnp.where(upd, t * (_TK // _L) + c, mcol)
    gmin = jnp.min(mval, axis=1, keepdims=True)
    kfull = mcol * _L + lax.broadcasted_iota(jnp.int32, (rows, _L), 1)
    ksel = jnp.where(mval == gmin, kfull, k_total)
    out_ref[...] = jnp.min(ksel, axis=1, keepdims=True)


def kernel(hidden_states, P, CB):
    B, T, D = hidden_states.shape
    NB, K, CD = CB.shape
    x = hidden_states.reshape(B * T, D)
    codes = pl.pallas_call(
        _vq_kernel,
        out_shape=jax.ShapeDtypeStruct((B * T, 1), jnp.int32),
    )(x, P[0], CB[0])
    return codes.reshape(B, NB, T)


# speedup vs baseline: 1.0696x; 1.0696x over previous
import jax
import jax.numpy as jnp
from jax import lax
from jax.experimental import pallas as pl
from jax.experimental.pallas import tpu as pltpu

_TK = 2048
_L = 128


def _vq_kernel(x_ref, p_ref, cb_ref, out_ref):
    rows = x_ref.shape[0]
    k_total = cb_ref.shape[0]
    cd = cb_ref.shape[1]

    h = jnp.dot(x_ref[...], p_ref[...], preferred_element_type=jnp.float32)
    n = jnp.sqrt(jnp.sum(h * h, axis=1, keepdims=True))
    hn = h / jnp.maximum(n, 1e-12)

    cbt_all = cb_ref[...].T

    mval = jnp.full((rows, _L), jnp.inf, dtype=jnp.float32)
    mcol = jnp.zeros((rows, _L), dtype=jnp.int32)
    for t in range(k_total // _TK):
        cbt = cbt_all[:, t * _TK:(t + 1) * _TK]
        cb2 = jnp.sum(cbt * cbt, axis=0, keepdims=True)
        cbt2 = cbt * (-2.0)
        s = jnp.dot(hn, cbt2, preferred_element_type=jnp.float32,
                    precision=lax.Precision.HIGHEST)
        d = cb2 + s
        for c in range(_TK // _L):
            dc = d[:, c * _L:(c + 1) * _L]
            upd = dc < mval
            mval = jnp.where(upd, dc, mval)
            mcol = jnp.where(upd, t * (_TK // _L) + c, mcol)
    gmin = jnp.min(mval, axis=1, keepdims=True)
    kfull = mcol * _L + lax.broadcasted_iota(jnp.int32, (rows, _L), 1)
    ksel = jnp.where(mval == gmin, kfull, k_total)
    out_ref[...] = jnp.min(ksel, axis=1, keepdims=True)


def kernel(hidden_states, P, CB):
    B, T, D = hidden_states.shape
    NB, K, CD = CB.shape
    x = hidden_states.reshape(B * T, D)
    codes = pl.pallas_call(
        _vq_kernel,
        out_shape=jax.ShapeDtypeStruct((B * T, 1), jnp.int32),
    )(x, P[0], CB[0])
    return codes.reshape(B, NB, T)
